# baseline (speedup 1.0000x reference)
import jax
import jax.numpy as jnp
from jax.experimental import pallas as pl
from jax.experimental.pallas import tpu as pltpu

_TS = 256
_TP = 512


def _gcn_kernel(a_ref, xt_ref, bd0_ref, bd1_ref, o_ref,
                abf_ref, g_ref, bdc_ref):
    t = pl.program_id(0)
    nt = pl.num_programs(0)
    npair = a_ref.shape[1] // _TP

    @pl.when(t == 0)
    def _weights():
        bdc_ref[...] = jnp.dot(bd0_ref[...], bd1_ref[...],
                               preferred_element_type=jnp.float32
                               ).astype(jnp.bfloat16)

    a_bf = a_ref[...].astype(jnp.bfloat16)
    for c in range(npair):
        abf_ref[c, pl.ds(t * _TS, _TS), :] = a_bf[:, c * _TP:(c + 1) * _TP]
    g_ref[pl.ds(t * _TS, _TS), :] = jnp.dot(
        a_bf, xt_ref[...], preferred_element_type=jnp.float32
    ).astype(jnp.bfloat16)

    @pl.when(t % 2 == 1)
    def _pairs():
        T = t // 2

        o_ref[pl.ds(T * _TP, _TP), :] = jnp.zeros((_TP, o_ref.shape[1]),
                                                  jnp.float32)

        def new_row(S, carry):
            o_ref[pl.ds(T * _TP, _TP), :] += jnp.dot(
                abf_ref[S, pl.ds(T * _TP, _TP), :],
                g_ref[pl.ds(S * _TP, _TP), :],
                preferred_element_type=jnp.float32)
            return carry

        jax.lax.fori_loop(0, T + 1, new_row, 0)

        def old_row(R, carry):
            o_ref[pl.ds(R * _TP, _TP), :] += jnp.dot(
                abf_ref[T, pl.ds(R * _TP, _TP), :],
                g_ref[pl.ds(T * _TP, _TP), :],
                preferred_element_type=jnp.float32)
            return carry

        jax.lax.fori_loop(0, T, old_row, 0)

    @pl.when(t == nt - 1)
    def _epilogue():
        for rb in range(npair):
            acc_bf = o_ref[rb * _TP:(rb + 1) * _TP, :].astype(jnp.bfloat16)
            o_ref[rb * _TP:(rb + 1) * _TP, :] = jnp.dot(
                acc_bf, bdc_ref[...], preferred_element_type=jnp.float32)


def kernel(x, adj, W0, W1):
    B, N, D = x.shape
    C = B * D
    S = N // _TS

    xt = jnp.transpose(x, (1, 0, 2)).reshape(N, C).astype(jnp.bfloat16)
    eye = jnp.eye(B, dtype=jnp.bfloat16)
    bd0 = jnp.kron(eye, W0.T.astype(jnp.bfloat16))
    bd1 = jnp.kron(eye, W1.T.astype(jnp.bfloat16))

    out_flat = pl.pallas_call(
        _gcn_kernel,
        grid=(S,),
        in_specs=[
            pl.BlockSpec((_TS, N), lambda t: (t, 0)),
            pl.BlockSpec((N, C), lambda t: (0, 0)),
            pl.BlockSpec((C, C), lambda t: (0, 0)),
            pl.BlockSpec((C, C), lambda t: (0, 0)),
        ],
        out_specs=pl.BlockSpec((N, C), lambda t: (0, 0)),
        out_shape=jax.ShapeDtypeStruct((N, C), jnp.float32),
        scratch_shapes=[
            pltpu.VMEM((N // _TP, N, _TP), jnp.bfloat16),
            pltpu.VMEM((N, C), jnp.bfloat16),
            pltpu.VMEM((C, C), jnp.bfloat16),
        ],
    )(adj, xt, bd0, bd1)

    return jnp.transpose(out_flat.reshape(N, B, D), (1, 0, 2))


# speedup vs baseline: 1.1345x; 1.1345x over previous
import jax
import jax.numpy as jnp
from jax.experimental import pallas as pl
from jax.experimental.pallas import tpu as pltpu

_TS = 256
_TP = 512
_DEPTH = 3


def _stripe_copy(a_hbm, ring_ref, sem_ref, stripe, slot):
    return pltpu.make_async_copy(
        a_hbm.at[pl.ds(stripe * _TS, _TS), :],
        ring_ref.at[slot],
        sem_ref.at[slot],
    )


def _gcn_kernel(a_hbm, xt_ref, bd0_ref, bd1_ref, o_ref,
                ring_ref, sem_ref, abf_ref, g_ref, bdc_ref):
    t = pl.program_id(0)
    nt = pl.num_programs(0)
    npair = a_hbm.shape[1] // _TP
    slot = jax.lax.rem(t, _DEPTH)

    @pl.when(t == 0)
    def _prologue():
        bdc_ref[...] = jnp.dot(bd0_ref[...], bd1_ref[...],
                               preferred_element_type=jnp.float32
                               ).astype(jnp.bfloat16)
        for s in range(_DEPTH):
            _stripe_copy(a_hbm, ring_ref, sem_ref, s, s).start()

    _stripe_copy(a_hbm, ring_ref, sem_ref, t, slot).wait()

    a_bf = ring_ref[slot].astype(jnp.bfloat16)
    for c in range(npair):
        abf_ref[c, pl.ds(t * _TS, _TS), :] = a_bf[:, c * _TP:(c + 1) * _TP]

    @pl.when(t + _DEPTH < nt)
    def _refill():
        _stripe_copy(a_hbm, ring_ref, sem_ref, t + _DEPTH, slot).start()

    g_ref[pl.ds(t * _TS, _TS), :] = jnp.dot(
        a_bf, xt_ref[...], preferred_element_type=jnp.float32
    ).astype(jnp.bfloat16)

    @pl.when(t % 2 == 1)
    def _pairs():
        T = t // 2

        o_ref[pl.ds(T * _TP, _TP), :] = jnp.zeros((_TP, o_ref.shape[1]),
                                                  jnp.float32)

        def new_row(S, carry):
            o_ref[pl.ds(T * _TP, _TP), :] += jnp.dot(
                abf_ref[S, pl.ds(T * _TP, _TP), :],
                g_ref[pl.ds(S * _TP, _TP), :],
                preferred_element_type=jnp.float32)
            return carry

        jax.lax.fori_loop(0, T + 1, new_row, 0)

        def old_row(R, carry):
            o_ref[pl.ds(R * _TP, _TP), :] += jnp.dot(
                abf_ref[T, pl.ds(R * _TP, _TP), :],
                g_ref[pl.ds(T * _TP, _TP), :],
                preferred_element_type=jnp.float32)
            return carry

        jax.lax.fori_loop(0, T, old_row, 0)

    @pl.when(t == nt - 1)
    def _epilogue():
        for rb in range(npair):
            acc_bf = o_ref[rb * _TP:(rb + 1) * _TP, :].astype(jnp.bfloat16)
            o_ref[rb * _TP:(rb + 1) * _TP, :] = jnp.dot(
                acc_bf, bdc_ref[...], preferred_element_type=jnp.float32)


def kernel(x, adj, W0, W1):
    B, N, D = x.shape
    C = B * D
    S = N // _TS

    xt = jnp.transpose(x, (1, 0, 2)).reshape(N, C).astype(jnp.bfloat16)
    eye = jnp.eye(B, dtype=jnp.bfloat16)
    bd0 = jnp.kron(eye, W0.T.astype(jnp.bfloat16))
    bd1 = jnp.kron(eye, W1.T.astype(jnp.bfloat16))

    out_flat = pl.pallas_call(
        _gcn_kernel,
        grid=(S,),
        in_specs=[
            pl.BlockSpec(memory_space=pltpu.MemorySpace.HBM),
            pl.BlockSpec((N, C), lambda t: (0, 0)),
            pl.BlockSpec((C, C), lambda t: (0, 0)),
            pl.BlockSpec((C, C), lambda t: (0, 0)),
        ],
        out_specs=pl.BlockSpec((N, C), lambda t: (0, 0)),
        out_shape=jax.ShapeDtypeStruct((N, C), jnp.float32),
        scratch_shapes=[
            pltpu.VMEM((_DEPTH, _TS, N), jnp.float32),
            pltpu.SemaphoreType.DMA((_DEPTH,)),
            pltpu.VMEM((N // _TP, N, _TP), jnp.bfloat16),
            pltpu.VMEM((N, C), jnp.bfloat16),
            pltpu.VMEM((C, C), jnp.bfloat16),
        ],
    )(adj, xt, bd0, bd1)

    return jnp.transpose(out_flat.reshape(N, B, D), (1, 0, 2))


# speedup vs baseline: 1.3562x; 1.1954x over previous
import jax
import jax.numpy as jnp
from jax.experimental import pallas as pl
from jax.experimental.pallas import tpu as pltpu

_TS = 256
_TP = 512
_DEPTH = 3
_CAP = 6


def _build_schedule(n_steps, n_blocks):
    ready = sorted(
        ((R, S) for R in range(n_blocks) for S in range(n_blocks)),
        key=lambda p: (2 * max(p) + 1, p))
    sched = [[] for _ in range(n_steps)]
    qi = 0
    for t in range(n_steps):
        cap = _CAP if t < n_steps - 1 else len(ready)
        while qi < len(ready) and len(sched[t]) < cap \
                and 2 * max(ready[qi]) + 1 <= t:
            sched[t].append(ready[qi])
            qi += 1
    return sched


def kernel(x, adj, W0, W1):
    B, N, D = x.shape
    C = B * D
    S = N // _TS
    NB = N // _TP
    sched = _build_schedule(S, NB)
    seen = set()
    first = [[(R not in seen, seen.add(R))[0] for (R, _s) in step]
             for step in sched]

    def stripe_copy(a_hbm, ring_ref, sem_ref, stripe, slot):
        return pltpu.make_async_copy(
            a_hbm.at[pl.ds(stripe * _TS, _TS), :],
            ring_ref.at[slot],
            sem_ref.at[slot],
        )

    def gcn_kernel(a_hbm, xt_ref, bd0_ref, bd1_ref, o_ref,
                   ring_ref, sem_ref, abf_ref, g_ref, bdc_ref):
        t = pl.program_id(0)
        slot = jax.lax.rem(t, _DEPTH)

        @pl.when(t == 0)
        def _prologue():
            bdc_ref[...] = jnp.dot(bd0_ref[...], bd1_ref[...],
                                   preferred_element_type=jnp.float32
                                   ).astype(jnp.bfloat16)
            for s in range(_DEPTH):
                stripe_copy(a_hbm, ring_ref, sem_ref, s, s).start()

        stripe_copy(a_hbm, ring_ref, sem_ref, t, slot).wait()

        a_bf = ring_ref[slot].astype(jnp.bfloat16)
        for c in range(NB):
            abf_ref[c, pl.ds(t * _TS, _TS), :] = a_bf[:, c * _TP:(c + 1) * _TP]

        @pl.when(t + _DEPTH < S)
        def _refill():
            stripe_copy(a_hbm, ring_ref, sem_ref, t + _DEPTH, slot).start()

        g_ref[pl.ds(t * _TS, _TS), :] = jnp.dot(
            a_bf, xt_ref[...], preferred_element_type=jnp.float32
        ).astype(jnp.bfloat16)

        for t_static in range(S):
            if not sched[t_static]:
                continue

            @pl.when(t == t_static)
            def _pairs(_step=t_static):
                for (R, Sb), init in zip(sched[_step], first[_step]):
                    prod = jnp.dot(
                        abf_ref[Sb, R * _TP:(R + 1) * _TP, :],
                        g_ref[Sb * _TP:(Sb + 1) * _TP, :],
                        preferred_element_type=jnp.float32)
                    if init:
                        o_ref[R * _TP:(R + 1) * _TP, :] = prod
                    else:
                        o_ref[R * _TP:(R + 1) * _TP, :] += prod

        @pl.when(t == S - 1)
        def _epilogue():
            for rb in range(NB):
                acc_bf = o_ref[rb * _TP:(rb + 1) * _TP, :].astype(jnp.bfloat16)
                o_ref[rb * _TP:(rb + 1) * _TP, :] = jnp.dot(
                    acc_bf, bdc_ref[...], preferred_element_type=jnp.float32)

    xt = jnp.transpose(x, (1, 0, 2)).reshape(N, C).astype(jnp.bfloat16)
    eye = jnp.eye(B, dtype=jnp.bfloat16)
    bd0 = jnp.kron(eye, W0.T.astype(jnp.bfloat16))
    bd1 = jnp.kron(eye, W1.T.astype(jnp.bfloat16))

    out_flat = pl.pallas_call(
        gcn_kernel,
        grid=(S,),
        in_specs=[
            pl.BlockSpec(memory_space=pltpu.MemorySpace.HBM),
            pl.BlockSpec((N, C), lambda t: (0, 0)),
            pl.BlockSpec((C, C), lambda t: (0, 0)),
            pl.BlockSpec((C, C), lambda t: (0, 0)),
        ],
        out_specs=pl.BlockSpec((N, C), lambda t: (0, 0)),
        out_shape=jax.ShapeDtypeStruct((N, C), jnp.float32),
        scratch_shapes=[
            pltpu.VMEM((_DEPTH, _TS, N), jnp.float32),
            pltpu.SemaphoreType.DMA((_DEPTH,)),
            pltpu.VMEM((N // _TP, N, _TP), jnp.bfloat16),
            pltpu.VMEM((N, C), jnp.bfloat16),
            pltpu.VMEM((C, C), jnp.bfloat16),
        ],
    )(adj, xt, bd0, bd1)

    return jnp.transpose(out_flat.reshape(N, B, D), (1, 0, 2))


# speedup vs baseline: 1.3855x; 1.0216x over previous
import jax
import jax.numpy as jnp
from jax.experimental import pallas as pl
from jax.experimental.pallas import tpu as pltpu

_TS = 256
_TP = 512
_DEPTH = 3
_CAP = 8


def _build_schedule(n_steps, n_blocks):
    ready = sorted(
        ((R, S) for R in range(n_blocks) for S in range(n_blocks)),
        key=lambda p: (2 * max(p) + 1, p))
    sched = [[] for _ in range(n_steps)]
    qi = 0
    for t in range(n_steps):
        cap = _CAP if t < n_steps - 1 else len(ready)
        while qi < len(ready) and len(sched[t]) < cap \
                and 2 * max(ready[qi]) + 1 <= t:
            sched[t].append(ready[qi])
            qi += 1
    return sched


def kernel(x, adj, W0, W1):
    B, N, D = x.shape
    C = B * D
    S = N // _TS
    NB = N // _TP
    sched = _build_schedule(S, NB)
    seen = set()
    first = [[(R not in seen, seen.add(R))[0] for (R, _s) in step]
             for step in sched]

    def stripe_copy(a_hbm, ring_ref, sem_ref, stripe, slot):
        return pltpu.make_async_copy(
            a_hbm.at[pl.ds(stripe * _TS, _TS), :],
            ring_ref.at[slot],
            sem_ref.at[slot],
        )

    def gcn_kernel(a_hbm, xt_ref, bd0_ref, bd1_ref, o_ref,
                   ring_ref, sem_ref, abf_ref, g_ref, bdc_ref):
        t = pl.program_id(0)
        slot = jax.lax.rem(t, _DEPTH)

        @pl.when(t == 0)
        def _prologue():
            bdc_ref[...] = jnp.dot(bd0_ref[...], bd1_ref[...],
                                   preferred_element_type=jnp.float32
                                   ).astype(jnp.bfloat16)
            for s in range(_DEPTH):
                stripe_copy(a_hbm, ring_ref, sem_ref, s, s).start()

        stripe_copy(a_hbm, ring_ref, sem_ref, t, slot).wait()

        a_bf = ring_ref[slot].astype(jnp.bfloat16)
        for c in range(NB):
            abf_ref[c, pl.ds(t * _TS, _TS), :] = a_bf[:, c * _TP:(c + 1) * _TP]

        @pl.when(t + _DEPTH < S)
        def _refill():
            stripe_copy(a_hbm, ring_ref, sem_ref, t + _DEPTH, slot).start()

        g_ref[pl.ds(t * _TS, _TS), :] = jnp.dot(
            a_bf, xt_ref[...], preferred_element_type=jnp.float32
        ).astype(jnp.bfloat16)

        for t_static in range(S):
            if not sched[t_static]:
                continue

            @pl.when(t == t_static)
            def _pairs(_step=t_static):
                for (R, Sb), init in zip(sched[_step], first[_step]):
                    prod = jnp.dot(
                        abf_ref[Sb, R * _TP:(R + 1) * _TP, :],
                        g_ref[Sb * _TP:(Sb + 1) * _TP, :],
                        preferred_element_type=jnp.float32)
                    if init:
                        o_ref[R * _TP:(R + 1) * _TP, :] = prod
                    else:
                        o_ref[R * _TP:(R + 1) * _TP, :] += prod

        @pl.when(t == S - 1)
        def _epilogue():
            for rb in range(NB):
                acc_bf = o_ref[rb * _TP:(rb + 1) * _TP, :].astype(jnp.bfloat16)
                o_ref[rb * _TP:(rb + 1) * _TP, :] = jnp.dot(
                    acc_bf, bdc_ref[...], preferred_element_type=jnp.float32)

    xt = jnp.transpose(x, (1, 0, 2)).reshape(N, C).astype(jnp.bfloat16)
    eye = jnp.eye(B, dtype=jnp.bfloat16)
    bd0 = jnp.kron(eye, W0.T.astype(jnp.bfloat16))
    bd1 = jnp.kron(eye, W1.T.astype(jnp.bfloat16))

    out_flat = pl.pallas_call(
        gcn_kernel,
        grid=(S,),
        in_specs=[
            pl.BlockSpec(memory_space=pltpu.MemorySpace.HBM),
            pl.BlockSpec((N, C), lambda t: (0, 0)),
            pl.BlockSpec((C, C), lambda t: (0, 0)),
            pl.BlockSpec((C, C), lambda t: (0, 0)),
        ],
        out_specs=pl.BlockSpec((N, C), lambda t: (0, 0)),
        out_shape=jax.ShapeDtypeStruct((N, C), jnp.float32),
        scratch_shapes=[
            pltpu.VMEM((_DEPTH, _TS, N), jnp.float32),
            pltpu.SemaphoreType.DMA((_DEPTH,)),
            pltpu.VMEM((N // _TP, N, _TP), jnp.bfloat16),
            pltpu.VMEM((N, C), jnp.bfloat16),
            pltpu.VMEM((C, C), jnp.bfloat16),
        ],
    )(adj, xt, bd0, bd1)

    return jnp.transpose(out_flat.reshape(N, B, D), (1, 0, 2))
